# baseline (speedup 1.0000x reference)
import functools

import jax
import jax.numpy as jnp
from jax import lax
from jax.experimental import pallas as pl
from jax.experimental.pallas import tpu as pltpu
from jax.experimental.pallas import tpu_sc as plsc

_VOCAB = 1000
_HIDDEN = 2048
_BATCH = 4
_SEQ = 4096
_TOKENS = _BATCH * _SEQ
_NUM_WORKERS = 32
_TOK_PER_W = _TOKENS // _NUM_WORKERS
_CHUNK = 64
_NCHUNK = _TOK_PER_W // _CHUNK


def _table_body(emb_ref, w_ref, b_ref, out_ref):
    out_ref[...] = lax.dot_general(
        emb_ref[...], w_ref[...],
        dimension_numbers=(((1,), (1,)), ((), ())),
        preferred_element_type=jnp.float32,
    ) + b_ref[...]


def _build_table(embed_table, lm_w, lm_b2d):
    return pl.pallas_call(
        _table_body,
        out_shape=jax.ShapeDtypeStruct((_VOCAB, _VOCAB), jnp.float32),
    )(embed_table, lm_w, lm_b2d)


_sc_mesh = plsc.VectorSubcoreMesh(core_axis_name="c", subcore_axis_name="s")


@functools.partial(
    pl.kernel,
    mesh=_sc_mesh,
    out_type=jax.ShapeDtypeStruct((_TOKENS, _VOCAB), jnp.float32),
    scratch_types=[
        pltpu.VMEM((_TOK_PER_W,), jnp.int32),
        pltpu.VMEM((_CHUNK, _VOCAB), jnp.float32),
        pltpu.VMEM((_CHUNK, _VOCAB), jnp.float32),
        pltpu.SemaphoreType.DMA,
        pltpu.SemaphoreType.DMA,
    ],
    compiler_params=pltpu.CompilerParams(use_tc_tiling_on_sc=False),
)
def _gather_rows(table_hbm, ids_hbm, out_hbm, idx_v, buf0, buf1, sem0, sem1):
    wid = lax.axis_index("s") * 2 + lax.axis_index("c")
    base = wid * _TOK_PER_W
    pltpu.sync_copy(ids_hbm.at[pl.ds(base, _TOK_PER_W)], idx_v)

    bufs = (buf0, buf1)
    sems = (sem0, sem1)
    copies = [None, None]
    copies[0] = pltpu.async_copy(
        table_hbm.at[idx_v.at[pl.ds(0, _CHUNK)]], bufs[0], sems[0])
    for i in range(_NCHUNK):
        nxt = (i + 1) % 2
        if i + 1 < _NCHUNK:
            copies[nxt] = pltpu.async_copy(
                table_hbm.at[idx_v.at[pl.ds((i + 1) * _CHUNK, _CHUNK)]],
                bufs[nxt], sems[nxt])
        copies[i % 2].wait()
        pltpu.sync_copy(bufs[i % 2], out_hbm.at[pl.ds(base + i * _CHUNK, _CHUNK)])


def kernel(input_ids, embed_table, gates, lm_w, lm_b):
    del gates
    table = _build_table(embed_table, lm_w, lm_b.reshape(1, _VOCAB))
    ids = input_ids.reshape(_TOKENS).astype(jnp.int32)
    out = _gather_rows(table, ids)
    return out.reshape(_BATCH, _SEQ, _VOCAB)


# speedup vs baseline: 1.4886x; 1.4886x over previous
import functools

import jax
import jax.numpy as jnp
from jax import lax
from jax.experimental import pallas as pl
from jax.experimental.pallas import tpu as pltpu
from jax.experimental.pallas import tpu_sc as plsc

_VOCAB = 1000
_VPAD = 1024
_HIDDEN = 2048
_BATCH = 4
_SEQ = 4096
_TOKENS = _BATCH * _SEQ
_NUM_WORKERS = 32
_TOK_PER_W = _TOKENS // _NUM_WORKERS
_CHUNK = 32
_NCHUNK = _TOK_PER_W // _CHUNK


def _table_body(emb_ref, w_ref, b_ref, out_ref):
    out_ref[...] = lax.dot_general(
        emb_ref[...], w_ref[...],
        dimension_numbers=(((1,), (1,)), ((), ())),
        preferred_element_type=jnp.float32,
    ) + b_ref[...]


def _build_table(embed_table, lm_w_pad, lm_b_pad2d):
    return pl.pallas_call(
        _table_body,
        out_shape=jax.ShapeDtypeStruct((_VOCAB, _VPAD), jnp.float32),
    )(embed_table, lm_w_pad, lm_b_pad2d)


_sc_mesh = plsc.VectorSubcoreMesh(core_axis_name="c", subcore_axis_name="s")


@functools.partial(
    pl.kernel,
    mesh=_sc_mesh,
    out_type=jax.ShapeDtypeStruct((_TOKENS, _VPAD), jnp.float32),
    scratch_types=[
        pltpu.VMEM((_TOK_PER_W,), jnp.int32),
        pltpu.VMEM((_CHUNK, _VPAD), jnp.float32),
        pltpu.VMEM((_CHUNK, _VPAD), jnp.float32),
        pltpu.SemaphoreType.DMA,
        pltpu.SemaphoreType.DMA,
    ],
)
def _gather_rows(table_hbm, ids_hbm, out_hbm, idx_v, buf0, buf1, sem0, sem1):
    wid = lax.axis_index("s") * 2 + lax.axis_index("c")
    base = wid * _TOK_PER_W
    pltpu.sync_copy(ids_hbm.at[pl.ds(base, _TOK_PER_W)], idx_v)

    bufs = (buf0, buf1)
    sems = (sem0, sem1)
    copies = [None, None]
    copies[0] = pltpu.async_copy(
        table_hbm.at[idx_v.at[pl.ds(0, _CHUNK)]], bufs[0], sems[0])
    for i in range(_NCHUNK):
        nxt = (i + 1) % 2
        if i + 1 < _NCHUNK:
            copies[nxt] = pltpu.async_copy(
                table_hbm.at[idx_v.at[pl.ds((i + 1) * _CHUNK, _CHUNK)]],
                bufs[nxt], sems[nxt])
        copies[i % 2].wait()
        pltpu.sync_copy(bufs[i % 2], out_hbm.at[pl.ds(base + i * _CHUNK, _CHUNK)])


def kernel(input_ids, embed_table, gates, lm_w, lm_b):
    del gates
    lm_w_pad = jnp.pad(lm_w, ((0, _VPAD - _VOCAB), (0, 0)))
    lm_b_pad = jnp.pad(lm_b, (0, _VPAD - _VOCAB)).reshape(1, _VPAD)
    table = _build_table(embed_table, lm_w_pad, lm_b_pad)
    ids = input_ids.reshape(_TOKENS).astype(jnp.int32)
    out = _gather_rows(table, ids)
    return out[:, :_VOCAB].reshape(_BATCH, _SEQ, _VOCAB)


# speedup vs baseline: 1.5774x; 1.0597x over previous
import functools

import jax
import jax.numpy as jnp
from jax import lax
from jax.experimental import pallas as pl
from jax.experimental.pallas import tpu as pltpu
from jax.experimental.pallas import tpu_sc as plsc

_VOCAB = 1000
_VPAD = 1024
_HIDDEN = 2048
_BATCH = 4
_SEQ = 4096
_TOKENS = _BATCH * _SEQ
_NUM_WORKERS = 32
_TOK_PER_W = _TOKENS // _NUM_WORKERS
_CHUNK = 32
_NCHUNK = _TOK_PER_W // _CHUNK


def _table_body(emb_ref, w_ref, b_ref, out_ref):
    out_ref[...] = jnp.pad(
        lax.dot_general(
            emb_ref[...], w_ref[...],
            dimension_numbers=(((1,), (1,)), ((), ())),
            preferred_element_type=jnp.float32,
        ) + b_ref[...],
        ((0, 0), (0, _VPAD - _VOCAB)),
    )


def _build_table(embed_table, lm_w, lm_b2d):
    return pl.pallas_call(
        _table_body,
        out_shape=jax.ShapeDtypeStruct((_VOCAB, _VPAD), jnp.float32),
    )(embed_table, lm_w, lm_b2d)


_sc_mesh = plsc.VectorSubcoreMesh(core_axis_name="c", subcore_axis_name="s")


@functools.partial(
    pl.kernel,
    mesh=_sc_mesh,
    out_type=jax.ShapeDtypeStruct((_TOKENS, _VPAD), jnp.float32),
    scratch_types=[
        pltpu.VMEM((_TOK_PER_W,), jnp.int32),
        pltpu.VMEM((_CHUNK, _VPAD), jnp.float32),
        pltpu.VMEM((_CHUNK, _VPAD), jnp.float32),
        pltpu.SemaphoreType.DMA,
        pltpu.SemaphoreType.DMA,
    ],
)
def _gather_rows(table_hbm, ids_hbm, out_hbm, idx_v, buf0, buf1, sem0, sem1):
    wid = lax.axis_index("s") * 2 + lax.axis_index("c")
    base = wid * _TOK_PER_W
    pltpu.sync_copy(ids_hbm.at[pl.ds(base, _TOK_PER_W)], idx_v)

    bufs = (buf0, buf1)
    sems = (sem0, sem1)
    copies = [None, None]
    copies[0] = pltpu.async_copy(
        table_hbm.at[idx_v.at[pl.ds(0, _CHUNK)]], bufs[0], sems[0])
    for i in range(_NCHUNK):
        nxt = (i + 1) % 2
        if i + 1 < _NCHUNK:
            copies[nxt] = pltpu.async_copy(
                table_hbm.at[idx_v.at[pl.ds((i + 1) * _CHUNK, _CHUNK)]],
                bufs[nxt], sems[nxt])
        copies[i % 2].wait()
        pltpu.sync_copy(bufs[i % 2], out_hbm.at[pl.ds(base + i * _CHUNK, _CHUNK)])


def kernel(input_ids, embed_table, gates, lm_w, lm_b):
    del gates
    table = _build_table(embed_table, lm_w, lm_b.reshape(1, _VOCAB))
    ids = input_ids.reshape(_TOKENS).astype(jnp.int32)
    out = _gather_rows(table, ids)
    return out[:, :_VOCAB].reshape(_BATCH, _SEQ, _VOCAB)
